# Initial kernel scaffold; baseline (speedup 1.0000x reference)
import jax
import jax.numpy as jnp
from jax.experimental import pallas as pl


def kernel(x, pos, edge_attr, edge_index, batch, enc_W, enc_b, eW1, eb1, eW2, eb2, nW1, nb1, nW2, nb2, ln_g, ln_b):
    raise NotImplementedError("write your pallas kernel here")


# speedup vs baseline: 1.0312x; 1.0312x over previous
import jax
import jax.numpy as jnp
from jax.experimental import pallas as pl

L = 4


def _layernorm(x, g, b, eps=1e-5):
    m = jnp.mean(x, axis=-1, keepdims=True)
    v = jnp.var(x, axis=-1, keepdims=True)
    return (x - m) / jnp.sqrt(v + eps) * g + b


def kernel(x, pos, edge_attr, edge_index, batch, enc_W, enc_b, eW1, eb1, eW2, eb2, nW1, nb1, nW2, nb2, ln_g, ln_b):
    feats = jnp.concatenate([x, pos], axis=1)
    h = feats @ enc_W + enc_b
    row = edge_index[0]
    col = edge_index[1]
    n_nodes = h.shape[0]
    coord_diff = pos[row] - pos[col]
    radial = jnp.sum(coord_diff ** 2, axis=1, keepdims=True)
    for i in range(L):
        identity = h
        A = h @ eW1[i, :32] + eb1[i]
        B = h @ eW1[i, 32:64]
        g = A[row] + B[col] + radial * eW1[i, 64]
        ef = jax.nn.relu(g)
        ef = jax.nn.relu(ef @ eW2[i] + eb2[i])
        agg = jax.ops.segment_sum(ef, row, num_segments=n_nodes)
        n_in = jnp.concatenate([h, agg], axis=1)
        out = jax.nn.relu(n_in @ nW1[i] + nb1[i])
        out = out @ nW2[i] + nb2[i]
        h = jax.nn.relu(_layernorm(out, ln_g[i], ln_b[i]))
        h = h + identity
    return h


# speedup vs baseline: 2.4380x; 2.3642x over previous
import functools

import jax
import jax.numpy as jnp
from jax import lax
from jax.experimental import pallas as pl
from jax.experimental.pallas import tpu as pltpu
from jax.experimental.pallas import tpu_sc as plsc

N = 50000
E = 800000
H = 32
NLAYERS = 4

NC = 2
NS = 16
NW = NC * NS
CB = 1024
NSTEP = 25
EPAD = NW * CB * NSTEP
NPAD = 51200
RPS = NPAD // NS
DUMMY = N

BN = 256
BE = 2048
f32 = jnp.float32

_mesh = plsc.VectorSubcoreMesh(core_axis_name="c", subcore_axis_name="s")
_sc_params = pltpu.CompilerParams(use_tc_tiling_on_sc=False)
_sc_params_scatter = pltpu.CompilerParams(
    use_tc_tiling_on_sc=False, internal_scratch_in_bytes=0)


def _make_gather(W):
    nv = W // 16
    nblk = CB // 128

    @functools.partial(
        pl.kernel,
        out_type=jax.ShapeDtypeStruct((EPAD, W), f32),
        mesh=_mesh,
        compiler_params=_sc_params,
        scratch_types=[
            pltpu.VMEM((nblk, 128), jnp.int32),
            pltpu.VMEM((nblk, 128), jnp.int32),
            pltpu.VMEM((CB, W), f32),
            pltpu.VMEM((CB, W), f32),
            pltpu.SemaphoreType.DMA,
            pltpu.SemaphoreType.DMA,
            pltpu.SemaphoreType.DMA,
        ],
    )
    def gather(atbl, btbl, rowi, coli, out, idxr, idxc, bufa, bufb,
               sem_i, sem_a, sem_b):
        cid = lax.axis_index("c")
        sid = lax.axis_index("s")
        wid = sid * NC + cid

        def step_fn(step, carry):
            rbase = (wid * NSTEP + step) * nblk
            cp1 = pltpu.async_copy(rowi.at[pl.ds(rbase, nblk)], idxr, sem_i)
            cp2 = pltpu.async_copy(coli.at[pl.ds(rbase, nblk)], idxc, sem_i)
            cp1.wait()
            cp2.wait()
            ga = [pltpu.async_copy(atbl.at[idxr.at[j]],
                                   bufa.at[pl.ds(j * 128, 128)], sem_a)
                  for j in range(nblk)]
            gb = [pltpu.async_copy(btbl.at[idxc.at[j]],
                                   bufb.at[pl.ds(j * 128, 128)], sem_b)
                  for j in range(nblk)]
            for cp in ga:
                cp.wait()
            for cp in gb:
                cp.wait()

            def add_fn(r, c2):
                for k in range(nv):
                    sl = pl.ds(k * 16, 16)
                    bufa[r, sl] = bufa[r, sl] + bufb[r, sl]
                return c2

            lax.fori_loop(0, CB, add_fn, 0)
            pltpu.sync_copy(bufa, out.at[pl.ds((wid * NSTEP + step) * CB, CB)])
            return carry

        lax.fori_loop(0, NSTEP, step_fn, 0)

    return gather


_gather48 = _make_gather(48)
_gather32 = _make_gather(32)


HALF = NPAD // NC
ACC_ROWS = 26112
ZR = ACC_ROWS // NS
OR_ = HALF // NS
SPS = EPAD // NS
NSTEP2 = SPS // CB


@functools.partial(
    pl.kernel,
    out_type=jax.ShapeDtypeStruct((NPAD, H), f32),
    mesh=_mesh,
    compiler_params=_sc_params_scatter,
    scratch_types=[
        pltpu.VMEM((CB // 128, 128), jnp.int32),
        pltpu.VMEM((CB // 128, 128), jnp.int32),
        pltpu.VMEM((CB, H), f32),
        pltpu.VMEM_SHARED((ACC_ROWS, H), f32),
        pltpu.SemaphoreType.DMA,
        pltpu.SemaphoreType.DMA,
    ],
)
def _scatter(ef, rowi, zrows, pout, idxr, idxl, bufe, acc, sem_i, sem_e):
    nblk = CB // 128
    cid = lax.axis_index("c")
    sid = lax.axis_index("s")
    base0 = cid * HALF

    pltpu.sync_copy(zrows, acc.at[pl.ds(sid * ZR, ZR)])
    plsc.subcore_barrier()

    def step_fn(step, carry):
        ebase = sid * SPS + step * CB
        cp1 = pltpu.async_copy(rowi.at[pl.ds(ebase // 128, nblk)], idxr, sem_i)
        cp2 = pltpu.async_copy(ef.at[pl.ds(ebase, CB)], bufe, sem_e)
        cp1.wait()
        cp2.wait()

        def loc_fn(t, c2):
            j = t // 8
            cc = (t % 8) * 16
            v = idxr[j, pl.ds(cc, 16)] - base0
            bad = (v < 0) | (v >= HALF)
            idxl[j, pl.ds(cc, 16)] = jnp.where(bad, HALF, v)
            return c2

        lax.fori_loop(0, nblk * 8, loc_fn, 0)
        for j in range(nblk):
            pltpu.sync_copy(bufe.at[pl.ds(j * 128, 128)],
                            acc.at[idxl.at[j]], add=True)
        return carry

    lax.fori_loop(0, NSTEP2, step_fn, 0)
    plsc.subcore_barrier()
    pltpu.sync_copy(acc.at[pl.ds(sid * OR_, OR_)],
                    pout.at[pl.ds(base0 + sid * OR_, OR_)])


def _enc_call(feats, pos4, encW, enc_b, eW1a, eW1b, eb1):
    def body(f_ref, p_ref, w_ref, b_ref, wa_ref, wb_ref, b1_ref,
             h_ref, a_ref, bt_ref):
        h = f_ref[:] @ w_ref[:] + b_ref[:]
        h_ref[:] = h
        a = h @ wa_ref[:] + b1_ref[:]
        b = h @ wb_ref[:]
        p = p_ref[:]
        z = jnp.zeros((BN, 12), f32)
        a_ref[:] = jnp.concatenate([a, p, z], axis=1)
        bt_ref[:] = jnp.concatenate([b, -p, z], axis=1)

    return pl.pallas_call(
        body,
        grid=(NPAD // BN,),
        in_specs=[
            pl.BlockSpec((BN, 40), lambda i: (i, 0)),
            pl.BlockSpec((BN, 4), lambda i: (i, 0)),
            pl.BlockSpec((40, H), lambda i: (0, 0)),
            pl.BlockSpec((1, H), lambda i: (0, 0)),
            pl.BlockSpec((H, H), lambda i: (0, 0)),
            pl.BlockSpec((H, H), lambda i: (0, 0)),
            pl.BlockSpec((1, H), lambda i: (0, 0)),
        ],
        out_specs=[
            pl.BlockSpec((BN, H), lambda i: (i, 0)),
            pl.BlockSpec((BN, 48), lambda i: (i, 0)),
            pl.BlockSpec((BN, 48), lambda i: (i, 0)),
        ],
        out_shape=[
            jax.ShapeDtypeStruct((NPAD, H), f32),
            jax.ShapeDtypeStruct((NPAD, 48), f32),
            jax.ShapeDtypeStruct((NPAD, 48), f32),
        ],
    )(feats, pos4, encW, enc_b, eW1a, eW1b, eb1)


def _edge0_call(s48, w2, b2, w65):
    def body(s_ref, w2_ref, b2_ref, w65_ref, ef_ref, r_ref):
        s = s_ref[:]
        g = s[:, :H]
        d = s[:, H:H + 4]
        r = jnp.sum(d * d, axis=1, keepdims=True)
        ef1 = jax.nn.relu(g + r * w65_ref[:])
        ef_ref[:] = jax.nn.relu(ef1 @ w2_ref[:] + b2_ref[:])
        r_ref[:] = r

    return pl.pallas_call(
        body,
        grid=(EPAD // BE,),
        in_specs=[
            pl.BlockSpec((BE, 48), lambda i: (i, 0)),
            pl.BlockSpec((H, H), lambda i: (0, 0)),
            pl.BlockSpec((1, H), lambda i: (0, 0)),
            pl.BlockSpec((1, H), lambda i: (0, 0)),
        ],
        out_specs=[
            pl.BlockSpec((BE, H), lambda i: (i, 0)),
            pl.BlockSpec((BE, 1), lambda i: (i, 0)),
        ],
        out_shape=[
            jax.ShapeDtypeStruct((EPAD, H), f32),
            jax.ShapeDtypeStruct((EPAD, 1), f32),
        ],
    )(s48, w2, b2, w65)


def _edge_call(g, radial, w2, b2, w65):
    def body(g_ref, r_ref, w2_ref, b2_ref, w65_ref, ef_ref):
        r = r_ref[:]
        ef1 = jax.nn.relu(g_ref[:] + r * w65_ref[:])
        ef_ref[:] = jax.nn.relu(ef1 @ w2_ref[:] + b2_ref[:])

    return pl.pallas_call(
        body,
        grid=(EPAD // BE,),
        in_specs=[
            pl.BlockSpec((BE, H), lambda i: (i, 0)),
            pl.BlockSpec((BE, 1), lambda i: (i, 0)),
            pl.BlockSpec((H, H), lambda i: (0, 0)),
            pl.BlockSpec((1, H), lambda i: (0, 0)),
            pl.BlockSpec((1, H), lambda i: (0, 0)),
        ],
        out_specs=pl.BlockSpec((BE, H), lambda i: (i, 0)),
        out_shape=jax.ShapeDtypeStruct((EPAD, H), f32),
    )(g, radial, w2, b2, w65)


def _node_call(h, p0, w1, b1, w2, b2, lg, lb, wa, wb, eb, last):
    def body(h_ref, p0_ref, w1_ref, b1_ref, w2_ref, b2_ref,
             g_ref, be_ref, wa_ref, wb_ref, ebi_ref, *outs):
        h_in = h_ref[:]
        agg = p0_ref[:]
        nin = jnp.concatenate([h_in, agg], axis=1)
        o = jax.nn.relu(nin @ w1_ref[:] + b1_ref[:])
        o = o @ w2_ref[:] + b2_ref[:]
        m = jnp.mean(o, axis=1, keepdims=True)
        v = jnp.mean((o - m) ** 2, axis=1, keepdims=True)
        ln = (o - m) / jnp.sqrt(v + 1e-5) * g_ref[:] + be_ref[:]
        hn = jax.nn.relu(ln) + h_in
        outs[0][:] = hn
        if not last:
            outs[1][:] = hn @ wa_ref[:] + ebi_ref[:]
            outs[2][:] = hn @ wb_ref[:]

    n_out = 1 if last else 3
    out_specs = [pl.BlockSpec((BN, H), lambda i: (i, 0)) for _ in range(n_out)]
    out_shape = [jax.ShapeDtypeStruct((NPAD, H), f32) for _ in range(n_out)]
    res = pl.pallas_call(
        body,
        grid=(NPAD // BN,),
        in_specs=[
            pl.BlockSpec((BN, H), lambda i: (i, 0)),
            pl.BlockSpec((BN, H), lambda i: (i, 0)),
            pl.BlockSpec((2 * H, H), lambda i: (0, 0)),
            pl.BlockSpec((1, H), lambda i: (0, 0)),
            pl.BlockSpec((H, H), lambda i: (0, 0)),
            pl.BlockSpec((1, H), lambda i: (0, 0)),
            pl.BlockSpec((1, H), lambda i: (0, 0)),
            pl.BlockSpec((1, H), lambda i: (0, 0)),
            pl.BlockSpec((H, H), lambda i: (0, 0)),
            pl.BlockSpec((H, H), lambda i: (0, 0)),
            pl.BlockSpec((1, H), lambda i: (0, 0)),
        ],
        out_specs=out_specs,
        out_shape=out_shape,
    )(h, p0, w1, b1, w2, b2, lg, lb, wa, wb, eb)
    return res


def kernel(x, pos, edge_attr, edge_index, batch, enc_W, enc_b, eW1, eb1,
           eW2, eb2, nW1, nb1, nW2, nb2, ln_g, ln_b):
    row = edge_index[0]
    col = edge_index[1]
    pad = jnp.full((EPAD - E,), DUMMY, jnp.int32)
    row2d = jnp.concatenate([row, pad]).reshape(EPAD // 128, 128)
    col2d = jnp.concatenate([col, pad]).reshape(EPAD // 128, 128)

    feats = jnp.zeros((NPAD, 40), f32)
    feats = feats.at[:N, :32].set(x).at[:N, 32:35].set(pos)
    encWp = jnp.zeros((40, H), f32).at[:35].set(enc_W)
    pos4 = jnp.zeros((NPAD, 4), f32).at[:N, :3].set(pos)
    zrows = jnp.zeros((ZR, H), f32)

    def r2(v):
        return v.reshape(1, H)

    h, atbl, btbl = _enc_call(feats, pos4, encWp, r2(enc_b),
                              eW1[0, :H], eW1[0, H:2 * H], r2(eb1[0]))

    radial = None
    for i in range(NLAYERS):
        if i == 0:
            s48 = _gather48(atbl, btbl, row2d, col2d)
            ef, radial = _edge0_call(s48, eW2[0], r2(eb2[0]), r2(eW1[0, 64]))
        else:
            g = _gather32(atbl, btbl, row2d, col2d)
            ef = _edge_call(g, radial, eW2[i], r2(eb2[i]), r2(eW1[i, 64]))
        agg = _scatter(ef, row2d, zrows)
        last = i == NLAYERS - 1
        if last:
            wa = wb = eW1[0, :H]
            eb = r2(eb1[0])
        else:
            wa = eW1[i + 1, :H]
            wb = eW1[i + 1, H:2 * H]
            eb = r2(eb1[i + 1])
        res = _node_call(h, agg, nW1[i], r2(nb1[i]), nW2[i], r2(nb2[i]),
                         r2(ln_g[i]), r2(ln_b[i]), wa, wb, eb, last)
        if last:
            h = res[0]
        else:
            h, atbl, btbl = res
    return h[:N]


# speedup vs baseline: 2.5282x; 1.0370x over previous
import functools

import jax
import jax.numpy as jnp
from jax import lax
from jax.experimental import pallas as pl
from jax.experimental.pallas import tpu as pltpu
from jax.experimental.pallas import tpu_sc as plsc

N = 50000
E = 800000
H = 32
NLAYERS = 4

NC = 2
NS = 16
NW = NC * NS
CB = 1024
NSTEP = 25
EPAD = NW * CB * NSTEP
NPAD = 51200
RPS = NPAD // NS
DUMMY = N

BN = 256
BE = 2048
f32 = jnp.float32

_mesh = plsc.VectorSubcoreMesh(core_axis_name="c", subcore_axis_name="s")
_sc_params = pltpu.CompilerParams(use_tc_tiling_on_sc=False)
_sc_params_scatter = pltpu.CompilerParams(
    use_tc_tiling_on_sc=False, internal_scratch_in_bytes=0)


EPW = EPAD // NW
ROWS = EPW // 128


def _make_gather(W, cb=256):
    nv = W // 16
    nblk = cb // 128
    nstep = EPW // cb
    nsup = nstep // 4

    @functools.partial(
        pl.kernel,
        out_type=jax.ShapeDtypeStruct((EPAD, W), f32),
        mesh=_mesh,
        compiler_params=_sc_params,
        scratch_types=[
            pltpu.VMEM((4, nblk, 128), jnp.int32),
            pltpu.VMEM((4, nblk, 128), jnp.int32),
            pltpu.VMEM((2, cb, W), f32),
            pltpu.VMEM((2, cb, W), f32),
        ] + [pltpu.SemaphoreType.DMA] * 8,
    )
    def gather(atbl, btbl, rowi, coli, out, idxr, idxc, bufa, bufb,
               si0, si1, si2, si3, sg0, sg1, so0, so1):
        cid = lax.axis_index("c")
        sid = lax.axis_index("s")
        wid = sid * NC + cid
        sem_i = (si0, si1, si2, si3)
        sem_g = (sg0, sg1)
        sem_o = (so0, so1)

        def fire_idx(g, q):
            src = pl.ds((wid * nstep + g) * nblk, nblk)
            pltpu.async_copy(rowi.at[src], idxr.at[q], sem_i[q])
            pltpu.async_copy(coli.at[src], idxc.at[q], sem_i[q])

        def wait_idx(q):
            pltpu.make_async_copy(rowi.at[pl.ds(0, nblk)], idxr.at[q],
                                  sem_i[q]).wait()
            pltpu.make_async_copy(coli.at[pl.ds(0, nblk)], idxc.at[q],
                                  sem_i[q]).wait()

        def fire(g, p, q):
            for j in range(nblk):
                dsl = pl.ds(j * 128, 128)
                pltpu.async_copy(atbl.at[idxr.at[q, j]],
                                 bufa.at[p, dsl], sem_g[p])
                pltpu.async_copy(btbl.at[idxc.at[q, j]],
                                 bufb.at[p, dsl], sem_g[p])

        def wait_gathers(p):
            for j in range(nblk):
                dsl = pl.ds(j * 128, 128)
                pltpu.make_async_copy(atbl.at[pl.ds(0, 128)],
                                      bufa.at[p, dsl], sem_g[p]).wait()
                pltpu.make_async_copy(btbl.at[pl.ds(0, 128)],
                                      bufb.at[p, dsl], sem_g[p]).wait()

        def add_compute(p):
            def add_fn(r, c2):
                for k in range(nv):
                    sl = pl.ds(k * 16, 16)
                    bufa[p, r, sl] = bufa[p, r, sl] + bufb[p, r, sl]
                return c2
            lax.fori_loop(0, cb, add_fn, 0, unroll=8)

        def fire_out(g, p):
            pltpu.async_copy(bufa.at[p],
                             out.at[pl.ds(wid * EPW + g * cb, cb)], sem_o[p])

        def wait_out(p):
            pltpu.make_async_copy(bufa.at[p], out.at[pl.ds(0, cb)],
                                  sem_o[p]).wait()

        fire_idx(0, 0)
        fire_idx(1, 1)

        def body(s, carry):
            for u in range(4):
                g = 4 * s + u
                p = u & 1
                qn = (u + 2) % 4
                if u < 2:
                    fire_idx(g + 2, qn)
                else:
                    @pl.when(s < nsup - 1)
                    def _():
                        fire_idx(g + 2, qn)
                wait_idx(u)
                if u < 2:
                    @pl.when(s >= 1)
                    def _():
                        wait_out(p)
                else:
                    wait_out(p)
                fire(g, p, u)
                if u == 0:
                    @pl.when(s >= 1)
                    def _():
                        wait_gathers(1 - p)
                        add_compute(1 - p)
                        fire_out(g - 1, 1 - p)
                else:
                    wait_gathers(1 - p)
                    add_compute(1 - p)
                    fire_out(g - 1, 1 - p)
            return carry

        lax.fori_loop(0, nsup, body, 0)
        wait_gathers(1)
        add_compute(1)
        fire_out(nstep - 1, 1)
        wait_out(0)
        wait_out(1)

    return gather


_gather48 = _make_gather(48)
_gather32 = _make_gather(32)


HALF = NPAD // NC
ACC_ROWS = 26112
ZR = ACC_ROWS // NS
OR_ = HALF // NS
SPS = EPAD // NS
NSTEP2 = SPS // CB


SCB = 128
SROWS = SPS // 128
SSTEP = SPS // SCB
SRING = 4
SSUP = SSTEP // SRING


@functools.partial(
    pl.kernel,
    out_type=jax.ShapeDtypeStruct((NPAD, H), f32),
    mesh=_mesh,
    compiler_params=_sc_params_scatter,
    scratch_types=[
        pltpu.VMEM((SROWS, 128), jnp.int32),
        pltpu.VMEM((SRING, SCB, H), f32),
        pltpu.VMEM_SHARED((ACC_ROWS, H), f32),
        pltpu.SemaphoreType.DMA,
        pltpu.SemaphoreType.DMA,
        pltpu.SemaphoreType.DMA,
        pltpu.SemaphoreType.DMA,
        pltpu.SemaphoreType.DMA,
        pltpu.SemaphoreType.DMA,
        pltpu.SemaphoreType.DMA,
        pltpu.SemaphoreType.DMA,
    ],
)
def _scatter(ef, rowi, zrows, pout, idxl, bufe, acc,
             se0, se1, se2, se3, ss0, ss1, ss2, ss3):
    cid = lax.axis_index("c")
    sid = lax.axis_index("s")
    base0 = cid * HALF
    sem_e = (se0, se1, se2, se3)
    sem_s = (ss0, ss1, ss2, ss3)

    pltpu.sync_copy(zrows, acc.at[pl.ds(sid * ZR, ZR)])
    pltpu.sync_copy(rowi.at[pl.ds(sid * SROWS, SROWS)], idxl)

    def loc_fn(t, c2):
        j = t // 8
        cc = (t % 8) * 16
        v = idxl[j, pl.ds(cc, 16)] - base0
        bad = (v < 0) | (v >= HALF)
        idxl[j, pl.ds(cc, 16)] = jnp.where(bad, HALF, v)
        return c2

    lax.fori_loop(0, SROWS * 8, loc_fn, 0, unroll=8)
    plsc.subcore_barrier()

    def fire_load(g, slot):
        pltpu.async_copy(ef.at[pl.ds(sid * SPS + g * SCB, SCB)],
                         bufe.at[slot], sem_e[slot])

    def wait_load(slot):
        pltpu.make_async_copy(ef.at[pl.ds(0, SCB)], bufe.at[slot],
                              sem_e[slot]).wait()

    def fire_scatter(g, slot):
        pltpu.async_copy(bufe.at[slot], acc.at[idxl.at[g]], sem_s[slot],
                         add=True)

    def wait_scatter(slot):
        pltpu.make_async_copy(bufe.at[slot], acc.at[pl.ds(0, SCB)],
                              sem_s[slot]).wait()

    fire_load(0, 0)
    fire_load(1, 1)

    def body(s, carry):
        for u in range(SRING):
            g = SRING * s + u
            nxt2 = (u + 2) % SRING
            if u < 2:
                @pl.when(s >= 1)
                def _():
                    wait_scatter(nxt2)
                fire_load(g + 2, nxt2)
            else:
                wait_scatter(nxt2)

                @pl.when(s < SSUP - 1)
                def _():
                    fire_load(g + 2, nxt2)
            wait_load(u)
            fire_scatter(g, u)
        return carry

    lax.fori_loop(0, SSUP, body, 0)
    wait_scatter(2)
    wait_scatter(3)
    plsc.subcore_barrier()
    pltpu.sync_copy(acc.at[pl.ds(sid * OR_, OR_)],
                    pout.at[pl.ds(base0 + sid * OR_, OR_)])


def _enc_call(feats, pos4, encW, enc_b, eW1a, eW1b, eb1):
    def body(f_ref, p_ref, w_ref, b_ref, wa_ref, wb_ref, b1_ref,
             h_ref, a_ref, bt_ref):
        h = f_ref[:] @ w_ref[:] + b_ref[:]
        h_ref[:] = h
        a = h @ wa_ref[:] + b1_ref[:]
        b = h @ wb_ref[:]
        p = p_ref[:]
        z = jnp.zeros((BN, 12), f32)
        a_ref[:] = jnp.concatenate([a, p, z], axis=1)
        bt_ref[:] = jnp.concatenate([b, -p, z], axis=1)

    return pl.pallas_call(
        body,
        grid=(NPAD // BN,),
        in_specs=[
            pl.BlockSpec((BN, 40), lambda i: (i, 0)),
            pl.BlockSpec((BN, 4), lambda i: (i, 0)),
            pl.BlockSpec((40, H), lambda i: (0, 0)),
            pl.BlockSpec((1, H), lambda i: (0, 0)),
            pl.BlockSpec((H, H), lambda i: (0, 0)),
            pl.BlockSpec((H, H), lambda i: (0, 0)),
            pl.BlockSpec((1, H), lambda i: (0, 0)),
        ],
        out_specs=[
            pl.BlockSpec((BN, H), lambda i: (i, 0)),
            pl.BlockSpec((BN, 48), lambda i: (i, 0)),
            pl.BlockSpec((BN, 48), lambda i: (i, 0)),
        ],
        out_shape=[
            jax.ShapeDtypeStruct((NPAD, H), f32),
            jax.ShapeDtypeStruct((NPAD, 48), f32),
            jax.ShapeDtypeStruct((NPAD, 48), f32),
        ],
    )(feats, pos4, encW, enc_b, eW1a, eW1b, eb1)


def _edge0_call(s48, w2, b2, w65):
    def body(s_ref, w2_ref, b2_ref, w65_ref, ef_ref, r_ref):
        s = s_ref[:]
        g = s[:, :H]
        d = s[:, H:H + 4]
        r = jnp.sum(d * d, axis=1, keepdims=True)
        ef1 = jax.nn.relu(g + r * w65_ref[:])
        ef_ref[:] = jax.nn.relu(ef1 @ w2_ref[:] + b2_ref[:])
        r_ref[:] = r

    return pl.pallas_call(
        body,
        grid=(EPAD // BE,),
        in_specs=[
            pl.BlockSpec((BE, 48), lambda i: (i, 0)),
            pl.BlockSpec((H, H), lambda i: (0, 0)),
            pl.BlockSpec((1, H), lambda i: (0, 0)),
            pl.BlockSpec((1, H), lambda i: (0, 0)),
        ],
        out_specs=[
            pl.BlockSpec((BE, H), lambda i: (i, 0)),
            pl.BlockSpec((BE, 1), lambda i: (i, 0)),
        ],
        out_shape=[
            jax.ShapeDtypeStruct((EPAD, H), f32),
            jax.ShapeDtypeStruct((EPAD, 1), f32),
        ],
    )(s48, w2, b2, w65)


def _edge_call(g, radial, w2, b2, w65):
    def body(g_ref, r_ref, w2_ref, b2_ref, w65_ref, ef_ref):
        r = r_ref[:]
        ef1 = jax.nn.relu(g_ref[:] + r * w65_ref[:])
        ef_ref[:] = jax.nn.relu(ef1 @ w2_ref[:] + b2_ref[:])

    return pl.pallas_call(
        body,
        grid=(EPAD // BE,),
        in_specs=[
            pl.BlockSpec((BE, H), lambda i: (i, 0)),
            pl.BlockSpec((BE, 1), lambda i: (i, 0)),
            pl.BlockSpec((H, H), lambda i: (0, 0)),
            pl.BlockSpec((1, H), lambda i: (0, 0)),
            pl.BlockSpec((1, H), lambda i: (0, 0)),
        ],
        out_specs=pl.BlockSpec((BE, H), lambda i: (i, 0)),
        out_shape=jax.ShapeDtypeStruct((EPAD, H), f32),
    )(g, radial, w2, b2, w65)


def _node_call(h, p0, w1, b1, w2, b2, lg, lb, wa, wb, eb, last):
    def body(h_ref, p0_ref, w1_ref, b1_ref, w2_ref, b2_ref,
             g_ref, be_ref, wa_ref, wb_ref, ebi_ref, *outs):
        h_in = h_ref[:]
        agg = p0_ref[:]
        nin = jnp.concatenate([h_in, agg], axis=1)
        o = jax.nn.relu(nin @ w1_ref[:] + b1_ref[:])
        o = o @ w2_ref[:] + b2_ref[:]
        m = jnp.mean(o, axis=1, keepdims=True)
        v = jnp.mean((o - m) ** 2, axis=1, keepdims=True)
        ln = (o - m) / jnp.sqrt(v + 1e-5) * g_ref[:] + be_ref[:]
        hn = jax.nn.relu(ln) + h_in
        outs[0][:] = hn
        if not last:
            outs[1][:] = hn @ wa_ref[:] + ebi_ref[:]
            outs[2][:] = hn @ wb_ref[:]

    n_out = 1 if last else 3
    out_specs = [pl.BlockSpec((BN, H), lambda i: (i, 0)) for _ in range(n_out)]
    out_shape = [jax.ShapeDtypeStruct((NPAD, H), f32) for _ in range(n_out)]
    res = pl.pallas_call(
        body,
        grid=(NPAD // BN,),
        in_specs=[
            pl.BlockSpec((BN, H), lambda i: (i, 0)),
            pl.BlockSpec((BN, H), lambda i: (i, 0)),
            pl.BlockSpec((2 * H, H), lambda i: (0, 0)),
            pl.BlockSpec((1, H), lambda i: (0, 0)),
            pl.BlockSpec((H, H), lambda i: (0, 0)),
            pl.BlockSpec((1, H), lambda i: (0, 0)),
            pl.BlockSpec((1, H), lambda i: (0, 0)),
            pl.BlockSpec((1, H), lambda i: (0, 0)),
            pl.BlockSpec((H, H), lambda i: (0, 0)),
            pl.BlockSpec((H, H), lambda i: (0, 0)),
            pl.BlockSpec((1, H), lambda i: (0, 0)),
        ],
        out_specs=out_specs,
        out_shape=out_shape,
    )(h, p0, w1, b1, w2, b2, lg, lb, wa, wb, eb)
    return res


def kernel(x, pos, edge_attr, edge_index, batch, enc_W, enc_b, eW1, eb1,
           eW2, eb2, nW1, nb1, nW2, nb2, ln_g, ln_b):
    row = edge_index[0]
    col = edge_index[1]
    pad = jnp.full((EPAD - E,), DUMMY, jnp.int32)
    row2d = jnp.concatenate([row, pad]).reshape(EPAD // 128, 128)
    col2d = jnp.concatenate([col, pad]).reshape(EPAD // 128, 128)

    feats = jnp.zeros((NPAD, 40), f32)
    feats = feats.at[:N, :32].set(x).at[:N, 32:35].set(pos)
    encWp = jnp.zeros((40, H), f32).at[:35].set(enc_W)
    pos4 = jnp.zeros((NPAD, 4), f32).at[:N, :3].set(pos)
    zrows = jnp.zeros((ZR, H), f32)

    def r2(v):
        return v.reshape(1, H)

    h, atbl, btbl = _enc_call(feats, pos4, encWp, r2(enc_b),
                              eW1[0, :H], eW1[0, H:2 * H], r2(eb1[0]))

    radial = None
    for i in range(NLAYERS):
        if i == 0:
            s48 = _gather48(atbl, btbl, row2d, col2d)
            ef, radial = _edge0_call(s48, eW2[0], r2(eb2[0]), r2(eW1[0, 64]))
        else:
            g = _gather32(atbl, btbl, row2d, col2d)
            ef = _edge_call(g, radial, eW2[i], r2(eb2[i]), r2(eW1[i, 64]))
        agg = _scatter(ef, row2d, zrows)
        last = i == NLAYERS - 1
        if last:
            wa = wb = eW1[0, :H]
            eb = r2(eb1[0])
        else:
            wa = eW1[i + 1, :H]
            wb = eW1[i + 1, H:2 * H]
            eb = r2(eb1[i + 1])
        res = _node_call(h, agg, nW1[i], r2(nb1[i]), nW2[i], r2(nb2[i]),
                         r2(ln_g[i]), r2(ln_b[i]), wa, wb, eb, last)
        if last:
            h = res[0]
        else:
            h, atbl, btbl = res
    return h[:N]
